# Initial kernel scaffold; baseline (speedup 1.0000x reference)
import jax
import jax.numpy as jnp
from jax.experimental import pallas as pl


def kernel(x, input_scale, W_probs, W_values, W_out, b_out, num_seqs):
    raise NotImplementedError("write your pallas kernel here")


# speedup vs baseline: 36.0785x; 36.0785x over previous
import functools

import jax
import jax.numpy as jnp
from jax.experimental import pallas as pl

_SEQ_LEN = 16
_NUM_SEQS = 2
_NUM_LEVELS = 128
_EPS = 1.2e-07
_BLK = 64


def _k1_body(x_ref, sc_ref, wp_ref, wv_ref, g0_ref, g1_ref, r_ref, ins_ref,
             idx_ref, w_ref, *, n_classes):
    blk = x_ref.shape[0]
    xs = x_ref[...] * sc_ref[0, 0]
    logits = jnp.dot(xs, wp_ref[...], preferred_element_type=jnp.float32)
    m = jnp.max(logits, axis=1, keepdims=True)
    e = jnp.exp(logits - m)
    p = e / jnp.sum(e, axis=1, keepdims=True)
    logp = jnp.log(p * (1.0 - n_classes * _EPS) + _EPS)
    lv = jnp.dot(xs, wv_ref[...], preferred_element_type=jnp.float32)
    iota = jax.lax.broadcasted_iota(jnp.int32, (blk, n_classes), 1)
    idx_cols = []
    lv_cols = []
    for g_ref in (g0_ref, g1_ref):
        keys = logp + g_ref[...]
        for _ in range(_SEQ_LEN):
            mx = jnp.max(keys, axis=1, keepdims=True)
            idx = jnp.min(jnp.where(keys >= mx, iota, n_classes), axis=1)
            hit = iota == idx[:, None]
            lv_cols.append(jnp.sum(jnp.where(hit, lv, 0.0), axis=1))
            keys = jnp.where(hit, -jnp.inf, keys)
            idx_cols.append(idx)
    idx_mat = jnp.stack(idx_cols, axis=1)
    lv_mat = jnp.stack(lv_cols, axis=1)
    r = r_ref[...]
    inv_ns = ins_ref[0, 0]
    w_parts = []
    for s in range(_NUM_SEQS):
        lv16 = lv_mat[:, s * _SEQ_LEN:(s + 1) * _SEQ_LEN]
        mx = jnp.max(lv16, axis=1, keepdims=True)
        ev = jnp.exp(lv16 - mx)
        cv = ev / jnp.sum(ev, axis=1, keepdims=True)
        t = cv * (_NUM_LEVELS - 1.0) + 0.999 * r[:, s * _SEQ_LEN:(s + 1) * _SEQ_LEN]
        disc = jnp.floor(t).astype(jnp.int32).astype(jnp.float32) * (
            1.0 / (_NUM_LEVELS - 1))
        w_parts.append(disc * inv_ns)
    idx_ref[...] = idx_mat
    w_ref[...] = jnp.concatenate(w_parts, axis=1)


def _k2_body(w_ref, idx_ref, wout_ref, b_ref, y_ref, *, n_classes):
    blk = w_ref.shape[0]
    w = w_ref[...]
    idx = idx_ref[...]
    iota = jax.lax.broadcasted_iota(jnp.int32, (blk, n_classes), 1)
    wd = jnp.zeros((blk, n_classes), jnp.float32)
    for j in range(_NUM_SEQS * _SEQ_LEN):
        wd = wd + jnp.where(iota == idx[:, j:j + 1], w[:, j:j + 1], 0.0)
    y = jnp.dot(wd, wout_ref[...], preferred_element_type=jnp.float32)
    y_ref[...] = y + b_ref[...]


def kernel(x, input_scale, W_probs, W_values, W_out, b_out, num_seqs):
    B, D = x.shape
    N = W_probs.shape[0]
    nblk = B // _BLK

    u = jax.random.uniform(jax.random.key(42), (B, _NUM_SEQS, N),
                           minval=1e-20, maxval=1.0)
    g = -jnp.log(-jnp.log(u))
    g0 = g[:, 0, :]
    g1 = g[:, 1, :]
    r = jax.random.uniform(jax.random.key(7), (B, _NUM_SEQS, _SEQ_LEN),
                           dtype=jnp.float32).reshape(B, _NUM_SEQS * _SEQ_LEN)

    sc2 = jnp.reshape(input_scale, (1, 1)).astype(jnp.float32)
    inv_ns = jnp.reshape(1.0 / jnp.asarray(num_seqs, jnp.float32), (1, 1))
    wpT = W_probs.T
    wvT = W_values.T
    woutT = W_out.T
    b2 = jnp.reshape(b_out, (1, D))

    nk = _NUM_SEQS * _SEQ_LEN
    idx_mat, w_mat = pl.pallas_call(
        functools.partial(_k1_body, n_classes=N),
        grid=(nblk,),
        in_specs=[
            pl.BlockSpec((_BLK, D), lambda i: (i, 0)),
            pl.BlockSpec((1, 1), lambda i: (0, 0)),
            pl.BlockSpec((D, N), lambda i: (0, 0)),
            pl.BlockSpec((D, N), lambda i: (0, 0)),
            pl.BlockSpec((_BLK, N), lambda i: (i, 0)),
            pl.BlockSpec((_BLK, N), lambda i: (i, 0)),
            pl.BlockSpec((_BLK, nk), lambda i: (i, 0)),
            pl.BlockSpec((1, 1), lambda i: (0, 0)),
        ],
        out_specs=[
            pl.BlockSpec((_BLK, nk), lambda i: (i, 0)),
            pl.BlockSpec((_BLK, nk), lambda i: (i, 0)),
        ],
        out_shape=[
            jax.ShapeDtypeStruct((B, nk), jnp.int32),
            jax.ShapeDtypeStruct((B, nk), jnp.float32),
        ],
    )(x, sc2, wpT, wvT, g0, g1, r, inv_ns)

    y = pl.pallas_call(
        functools.partial(_k2_body, n_classes=N),
        grid=(nblk,),
        in_specs=[
            pl.BlockSpec((_BLK, nk), lambda i: (i, 0)),
            pl.BlockSpec((_BLK, nk), lambda i: (i, 0)),
            pl.BlockSpec((N, D), lambda i: (0, 0)),
            pl.BlockSpec((1, D), lambda i: (0, 0)),
        ],
        out_specs=pl.BlockSpec((_BLK, D), lambda i: (i, 0)),
        out_shape=jax.ShapeDtypeStruct((B, D), jnp.float32),
    )(w_mat, idx_mat, woutT, b2)
    return y
